# baseline (speedup 1.0000x reference)
import functools

import jax
import jax.numpy as jnp
from jax import lax
from jax.experimental import pallas as pl
from jax.experimental.pallas import tpu as pltpu
from jax.experimental.pallas import tpu_sc as plsc

N = 50000
E = 800000
D = 64

NPAD = 51200
RPT = NPAD // 16
_W = 96
EPAD = 860160
S = EPAD // _W
SPT = S // 16
SPT_HALF = S // 32

_SC_MESH = plsc.VectorSubcoreMesh(core_axis_name="c", subcore_axis_name="s")
_SC_PARAMS = pltpu.CompilerParams(use_tc_tiling_on_sc=False)


_DG = 7
_DNGRP = SPT_HALF // _DG


@functools.partial(
    pl.kernel,
    out_type=jax.ShapeDtypeStruct((2, 16, RPT, 8), jnp.float32),
    mesh=_SC_MESH,
    compiler_params=_SC_PARAMS,
    scratch_types=[
        pltpu.VMEM((_W, 8), jnp.float32),
        pltpu.VMEM((2, _DG, _W), jnp.int32),
        pltpu.VMEM_SHARED((NPAD, 8), jnp.float32),
        pltpu.SemaphoreType.DMA,
    ],
)
def _deg_kernel(dst_hbm, zrows_hbm, upat_hbm, out_hbm, upat_v, didx, acc,
                ssem):
    c = lax.axis_index("c")
    s = lax.axis_index("s")
    pltpu.sync_copy(upat_hbm, upat_v)
    base = s * RPT
    pltpu.sync_copy(zrows_hbm, acc.at[pl.ds(base, RPT)])
    plsc.subcore_barrier()

    t0 = c * (S // 2) + s * SPT_HALF

    def stage(grp, b):
        pltpu.sync_copy(dst_hbm.at[pl.ds(t0 + grp * _DG, _DG)], didx.at[b])

    stage(0, 0)

    @pl.loop(0, _DNGRP, step=2)
    def _(g):
        for b in range(2):
            gg = g + b

            @pl.when(gg + 1 < _DNGRP)
            def _():
                stage(gg + 1, 1 - b)

            for j in range(_DG):
                pltpu.async_copy(upat_v, acc.at[didx.at[b, j]], ssem,
                                 add=True)
            for j in range(_DG):
                pltpu.make_async_copy(upat_v, acc.at[didx.at[b, j]],
                                      ssem).wait()

    plsc.subcore_barrier()
    pltpu.sync_copy(acc.at[pl.ds(base, RPT)], out_hbm.at[c, s])


_G = 4
_NGRP = SPT // _G


@functools.partial(
    pl.kernel,
    out_type=jax.ShapeDtypeStruct((NPAD, 2, 32), jnp.float32),
    mesh=_SC_MESH,
    compiler_params=_SC_PARAMS,
    scratch_types=[
        pltpu.VMEM((4, _G, _W), jnp.int32),
        pltpu.VMEM((4, _G, _W), jnp.int32),
        pltpu.VMEM((2, _G, _W, 32), jnp.float32),
        pltpu.VMEM_SHARED((NPAD, 32), jnp.float32),
        pltpu.SemaphoreType.DMA((4,)),
        pltpu.SemaphoreType.DMA((2,)),
        pltpu.SemaphoreType.DMA,
    ],
)
def _agg_kernel(hs_hbm, zrows_hbm, srcg_hbm, dst_hbm, out_hbm, sidx, didx,
                rows, acc, isem, gsem, ssem):
    c = lax.axis_index("c")
    s = lax.axis_index("s")
    base = s * RPT
    pltpu.sync_copy(zrows_hbm, acc.at[pl.ds(base, RPT)])
    plsc.subcore_barrier()

    t0 = s * SPT

    def stage_async(grp, ib):
        pltpu.async_copy(srcg_hbm.at[c, pl.ds(t0 + grp * _G, _G)],
                         sidx.at[ib], isem.at[ib])
        pltpu.async_copy(dst_hbm.at[pl.ds(t0 + grp * _G, _G)],
                         didx.at[ib], isem.at[ib])

    def wait_stage(grp, ib):
        pltpu.make_async_copy(srcg_hbm.at[c, pl.ds(t0 + grp * _G, _G)],
                              sidx.at[ib], isem.at[ib]).wait()
        pltpu.make_async_copy(dst_hbm.at[pl.ds(t0 + grp * _G, _G)],
                              didx.at[ib], isem.at[ib]).wait()

    def fire_gathers(ib, rb):
        for j in range(_G):
            pltpu.async_copy(hs_hbm.at[sidx.at[ib, j]], rows.at[rb, j],
                             gsem.at[rb])

    def wait_gathers(ib, rb):
        for j in range(_G):
            pltpu.make_async_copy(hs_hbm.at[sidx.at[ib, j]], rows.at[rb, j],
                                  gsem.at[rb]).wait()

    def fire_scatters(ib, rb):
        for j in range(_G):
            pltpu.async_copy(rows.at[rb, j], acc.at[didx.at[ib, j]], ssem,
                             add=True)

    def drain_scatters(ib, rb):
        for j in range(_G):
            pltpu.make_async_copy(rows.at[rb, j], acc.at[didx.at[ib, j]],
                                  ssem).wait()

    stage_async(0, 0)
    stage_async(1, 1)
    wait_stage(0, 0)
    fire_gathers(0, 0)

    @pl.loop(0, _NGRP, step=4)
    def _(g):
        for b in range(4):
            gg = g + b
            ib = b
            rb = b % 2
            ibp = (b - 1) % 4
            rbp = (b + 1) % 2

            @pl.when(gg >= 1)
            def _():
                drain_scatters(ibp, rbp)

            @pl.when(gg + 2 < _NGRP)
            def _():
                stage_async(gg + 2, (b + 2) % 4)

            @pl.when(gg + 1 < _NGRP)
            def _():
                wait_stage(gg + 1, (b + 1) % 4)
                fire_gathers((b + 1) % 4, (b + 1) % 2)

            wait_gathers(ib, rb)
            fire_scatters(ib, rb)

    drain_scatters(3, 1)
    plsc.subcore_barrier()
    pltpu.sync_copy(acc.at[pl.ds(base, RPT)],
                    out_hbm.at[pl.ds(base, RPT), c])


_BN = 512
_BR = _BN // 2
_GRID = (NPAD // _BN,)


def _pre_body(x_ref, degp_ref, w2_ref, sel_ref, hs_ref, dinvp_ref):
    dd = degp_ref[...]
    d0 = dd[0] + dd[1]
    dinvp = lax.rsqrt(jnp.maximum(
        jnp.dot(d0, sel_ref[...], preferred_element_type=jnp.float32), 1.0))
    h = jnp.dot(x_ref[...], w2_ref[...], preferred_element_type=jnp.float32)
    hs_ref[...] = h * dinvp
    dinvp_ref[...] = dinvp


def _pre_call(x2, degp2, w2, sel):
    return pl.pallas_call(
        _pre_body,
        grid=_GRID,
        in_specs=[
            pl.BlockSpec((_BR, 128), lambda i: (i, 0)),
            pl.BlockSpec((2, _BR, 16), lambda i: (0, i, 0)),
            pl.BlockSpec((128, 128), lambda i: (0, 0)),
            pl.BlockSpec((16, 128), lambda i: (0, 0)),
        ],
        out_specs=[
            pl.BlockSpec((_BR, 128), lambda i: (i, 0)),
            pl.BlockSpec((_BR, 128), lambda i: (i, 0)),
        ],
        out_shape=[
            jax.ShapeDtypeStruct((NPAD // 2, 128), jnp.float32),
            jax.ShapeDtypeStruct((NPAD // 2, 128), jnp.float32),
        ],
    )(x2, degp2, w2, sel)


def _make_post_body(has_prev, has_next):
    def body(*refs):
        agg_ref, dinvp_ref, m64_ref, b2_ref, g2_ref, beta2_ref = refs[:6]
        k = 6
        hprev_ref = None
        wn_ref = None
        if has_prev:
            hprev_ref = refs[k]
            k += 1
        if has_next:
            wn_ref = refs[k]
            k += 1
        h_ref = refs[k]
        k += 1
        hs_ref = refs[k] if has_next else None

        dinvp = dinvp_ref[...]
        del m64_ref
        v = agg_ref[...] * dinvp + b2_ref[0]
        mu_a = jnp.mean(v[:, :D], axis=1, keepdims=True)
        mu_b = jnp.mean(v[:, D:], axis=1, keepdims=True)
        mu = jnp.concatenate([jnp.broadcast_to(mu_a, (_BR, D)),
                              jnp.broadcast_to(mu_b, (_BR, D))], axis=1)
        xc = v - mu
        x2c = xc * xc
        var_a = jnp.mean(x2c[:, :D], axis=1, keepdims=True)
        var_b = jnp.mean(x2c[:, D:], axis=1, keepdims=True)
        var = jnp.concatenate([jnp.broadcast_to(var_a, (_BR, D)),
                               jnp.broadcast_to(var_b, (_BR, D))], axis=1)
        y = xc * lax.rsqrt(var + 1e-5) * g2_ref[0] + beta2_ref[0]
        y = jnp.where(y >= 0, y, 0.01 * y)
        hcur = 0.5 * y
        if has_prev:
            hcur = hcur + hprev_ref[...]
        h_ref[...] = hcur
        if has_next:
            hs_ref[...] = jnp.dot(
                y, wn_ref[...], preferred_element_type=jnp.float32) * dinvp

    return body


def _post_call(agg2, dinvp, m64, b2, g2, beta2, hprev, w2next):
    has_prev = hprev is not None
    has_next = w2next is not None
    in_specs = [
        pl.BlockSpec((_BR, 128), lambda i: (i, 0)),
        pl.BlockSpec((_BR, 128), lambda i: (i, 0)),
        pl.BlockSpec((128, 128), lambda i: (0, 0)),
        pl.BlockSpec((1, 128), lambda i: (0, 0)),
        pl.BlockSpec((1, 128), lambda i: (0, 0)),
        pl.BlockSpec((1, 128), lambda i: (0, 0)),
    ]
    args = [agg2, dinvp, m64, b2, g2, beta2]
    if has_prev:
        in_specs.append(pl.BlockSpec((_BR, 128), lambda i: (i, 0)))
        args.append(hprev)
    if has_next:
        in_specs.append(pl.BlockSpec((128, 128), lambda i: (0, 0)))
        args.append(w2next)
    out_specs = [pl.BlockSpec((_BR, 128), lambda i: (i, 0))]
    out_shape = [jax.ShapeDtypeStruct((NPAD // 2, 128), jnp.float32)]
    if has_next:
        out_specs.append(pl.BlockSpec((_BR, 128), lambda i: (i, 0)))
        out_shape.append(jax.ShapeDtypeStruct((NPAD // 2, 128), jnp.float32))
    res = pl.pallas_call(
        _make_post_body(has_prev, has_next),
        grid=_GRID,
        in_specs=in_specs,
        out_specs=out_specs,
        out_shape=out_shape,
    )(*args)
    return res if has_next else (res[0], None)


def kernel(x, edge_index, W0, b0, g0, beta0, W1, b1, g1, beta1,
           W2, b2, g2, beta2, W3, b3, g3, beta3):
    Ws = [W0, W1, W2, W3]
    bs = [b0, b1, b2, b3]
    gs = [g0, g1, g2, g3]
    betas = [beta0, beta1, beta2, beta3]

    src = edge_index[0]
    dst = edge_index[1]
    loop_ids = jnp.arange(N, dtype=jnp.int32)
    pad_ids = N + (jnp.arange(EPAD - E - N, dtype=jnp.int32) % (NPAD - N))
    src_p = jnp.concatenate([src, loop_ids, pad_ids])
    dst_p = jnp.concatenate([dst, loop_ids, pad_ids])
    srcg = jnp.stack([2 * src_p, 2 * src_p + 1]).reshape(2, S, _W)
    dsts = dst_p.reshape(S, _W)
    x2 = jnp.pad(x, ((0, NPAD - N), (0, 0))).reshape(NPAD // 2, 128)

    zero128 = jnp.zeros((128, 128), jnp.float32)
    w2s = [jnp.block([[w, jnp.zeros((D, D), w.dtype)],
                      [jnp.zeros((D, D), w.dtype), w]]) for w in Ws]
    ones64 = jnp.ones((D, D), jnp.float32)
    m64 = zero128.at[:D, :D].set(ones64).at[D:, D:].set(ones64)
    sel = jnp.zeros((16, 128), jnp.float32).at[0, :D].set(1.0).at[8, D:].set(1.0)
    b2s = [jnp.tile(b, 2).reshape(1, 128) for b in bs]
    g2s = [jnp.tile(g, 2).reshape(1, 128) for g in gs]
    beta2s = [jnp.tile(bb, 2).reshape(1, 128) for bb in betas]

    zrows = jnp.zeros((RPT, 8), jnp.float32)
    zrows32 = jnp.zeros((RPT, 32), jnp.float32)
    upat = jnp.zeros((_W, 8), jnp.float32).at[:, 0].set(1.0)
    degp2 = _deg_kernel(dsts, zrows, upat).reshape(2, NPAD // 2, 16)
    hs, dinvp = _pre_call(x2, degp2, w2s[0], sel)
    h = None
    for i in range(4):
        agg = _agg_kernel(hs.reshape(2 * NPAD, 32), zrows32, srcg, dsts)
        agg2 = agg.reshape(NPAD // 2, 128)
        w2next = w2s[i + 1] if i < 3 else None
        h, hs = _post_call(agg2, dinvp, m64, b2s[i], g2s[i], beta2s[i], h,
                           w2next)
    return h.reshape(NPAD, D)[:N]


# speedup vs baseline: 1.2310x; 1.2310x over previous
import functools

import jax
import jax.numpy as jnp
from jax import lax
from jax.experimental import pallas as pl
from jax.experimental.pallas import tpu as pltpu
from jax.experimental.pallas import tpu_sc as plsc

N = 50000
E = 800000
D = 64

NPAD = 51200
RPT = NPAD // 16
_W = 96
EPAD = 860160
S = EPAD // _W
SPT = S // 16
SPT_HALF = S // 32

_SC_MESH = plsc.VectorSubcoreMesh(core_axis_name="c", subcore_axis_name="s")
_SC_PARAMS = pltpu.CompilerParams(use_tc_tiling_on_sc=False)


_DG = 7
_DNGRP = SPT_HALF // _DG


@functools.partial(
    pl.kernel,
    out_type=jax.ShapeDtypeStruct((2, 16, RPT, 8), jnp.float32),
    mesh=_SC_MESH,
    compiler_params=_SC_PARAMS,
    scratch_types=[
        pltpu.VMEM((_W, 8), jnp.float32),
        pltpu.VMEM((2, _DG, _W), jnp.int32),
        pltpu.VMEM_SHARED((NPAD, 8), jnp.float32),
        pltpu.SemaphoreType.DMA,
    ],
)
def _deg_kernel(dst_hbm, zrows_hbm, upat_hbm, out_hbm, upat_v, didx, acc,
                ssem):
    c = lax.axis_index("c")
    s = lax.axis_index("s")
    pltpu.sync_copy(upat_hbm, upat_v)
    base = s * RPT
    pltpu.sync_copy(zrows_hbm, acc.at[pl.ds(base, RPT)])
    plsc.subcore_barrier()

    t0 = c * (S // 2) + s * SPT_HALF

    def stage(grp, b):
        pltpu.sync_copy(dst_hbm.at[pl.ds(t0 + grp * _DG, _DG)], didx.at[b])

    stage(0, 0)

    @pl.loop(0, _DNGRP, step=2)
    def _(g):
        for b in range(2):
            gg = g + b

            @pl.when(gg + 1 < _DNGRP)
            def _():
                stage(gg + 1, 1 - b)

            for j in range(_DG):
                pltpu.async_copy(upat_v, acc.at[didx.at[b, j]], ssem,
                                 add=True)
            for j in range(_DG):
                pltpu.make_async_copy(upat_v, acc.at[didx.at[b, j]],
                                      ssem).wait()

    plsc.subcore_barrier()
    pltpu.sync_copy(acc.at[pl.ds(base, RPT)], out_hbm.at[c, s])


_G = 4
_NGRP = SPT // _G


@functools.partial(
    pl.kernel,
    out_type=jax.ShapeDtypeStruct((NPAD, 2, 32), jnp.float32),
    mesh=_SC_MESH,
    compiler_params=_SC_PARAMS,
    scratch_types=[
        pltpu.VMEM((4, _G, _W), jnp.int32),
        pltpu.VMEM((4, _G, _W), jnp.int32),
        pltpu.VMEM((2, _G, _W, 32), jnp.float32),
        pltpu.VMEM_SHARED((NPAD, 32), jnp.float32),
        pltpu.SemaphoreType.DMA((4,)),
        pltpu.SemaphoreType.DMA((2,)),
        pltpu.SemaphoreType.DMA,
    ],
)
def _agg_kernel(hs_hbm, zrows_hbm, srcg_hbm, dst_hbm, out_hbm, sidx, didx,
                rows, acc, isem, gsem, ssem):
    c = lax.axis_index("c")
    s = lax.axis_index("s")
    base = s * RPT
    pltpu.sync_copy(zrows_hbm, acc.at[pl.ds(base, RPT)])
    plsc.subcore_barrier()

    t0 = s * SPT

    def stage_async(grp, ib):
        pltpu.async_copy(srcg_hbm.at[c, pl.ds(t0 + grp * _G, _G)],
                         sidx.at[ib], isem.at[ib])
        pltpu.async_copy(dst_hbm.at[pl.ds(t0 + grp * _G, _G)],
                         didx.at[ib], isem.at[ib])

    def wait_stage(grp, ib):
        pltpu.make_async_copy(srcg_hbm.at[c, pl.ds(t0 + grp * _G, _G)],
                              sidx.at[ib], isem.at[ib]).wait()
        pltpu.make_async_copy(dst_hbm.at[pl.ds(t0 + grp * _G, _G)],
                              didx.at[ib], isem.at[ib]).wait()

    def fire_gathers(ib, rb):
        for j in range(_G):
            pltpu.async_copy(hs_hbm.at[sidx.at[ib, j]], rows.at[rb, j],
                             gsem.at[rb])

    def wait_gathers(ib, rb):
        for j in range(_G):
            pltpu.make_async_copy(hs_hbm.at[sidx.at[ib, j]], rows.at[rb, j],
                                  gsem.at[rb]).wait()

    def fire_scatters(ib, rb):
        for j in range(_G):
            pltpu.async_copy(rows.at[rb, j], acc.at[didx.at[ib, j]], ssem,
                             add=True)

    def drain_scatters(ib, rb):
        for j in range(_G):
            pltpu.make_async_copy(rows.at[rb, j], acc.at[didx.at[ib, j]],
                                  ssem).wait()

    stage_async(0, 0)
    stage_async(1, 1)
    wait_stage(0, 0)
    fire_gathers(0, 0)

    @pl.loop(0, _NGRP, step=4)
    def _(g):
        for b in range(4):
            gg = g + b
            ib = b
            rb = b % 2
            ibp = (b - 1) % 4
            rbp = (b + 1) % 2

            @pl.when(gg >= 1)
            def _():
                drain_scatters(ibp, rbp)

            @pl.when(gg + 2 < _NGRP)
            def _():
                stage_async(gg + 2, (b + 2) % 4)

            @pl.when(gg + 1 < _NGRP)
            def _():
                wait_stage(gg + 1, (b + 1) % 4)
                fire_gathers((b + 1) % 4, (b + 1) % 2)

            wait_gathers(ib, rb)
            fire_scatters(ib, rb)

    drain_scatters(3, 1)
    plsc.subcore_barrier()
    pltpu.sync_copy(acc.at[pl.ds(base, RPT)],
                    out_hbm.at[pl.ds(base, RPT), c])


_BN = 2048
_BR = _BN // 2
_GRID = (NPAD // _BN,)


def _pre_body(x_ref, degp_ref, w2_ref, sel_ref, hs_ref, dinvp_ref):
    dd = degp_ref[...]
    d0 = dd[0] + dd[1]
    dinvp = lax.rsqrt(jnp.maximum(
        jnp.dot(d0, sel_ref[...], preferred_element_type=jnp.float32), 1.0))
    h = jnp.dot(x_ref[...], w2_ref[...], preferred_element_type=jnp.float32)
    hs_ref[...] = h * dinvp
    dinvp_ref[...] = dinvp


def _pre_call(x2, degp2, w2, sel):
    return pl.pallas_call(
        _pre_body,
        grid=_GRID,
        in_specs=[
            pl.BlockSpec((_BR, 128), lambda i: (i, 0)),
            pl.BlockSpec((2, _BR, 16), lambda i: (0, i, 0)),
            pl.BlockSpec((128, 128), lambda i: (0, 0)),
            pl.BlockSpec((16, 128), lambda i: (0, 0)),
        ],
        out_specs=[
            pl.BlockSpec((_BR, 128), lambda i: (i, 0)),
            pl.BlockSpec((_BR, 128), lambda i: (i, 0)),
        ],
        out_shape=[
            jax.ShapeDtypeStruct((NPAD // 2, 128), jnp.float32),
            jax.ShapeDtypeStruct((NPAD // 2, 128), jnp.float32),
        ],
    )(x2, degp2, w2, sel)


def _make_post_body(has_prev, has_next):
    def body(*refs):
        agg_ref, dinvp_ref, m64_ref, b2_ref, g2_ref, beta2_ref = refs[:6]
        k = 6
        hprev_ref = None
        wn_ref = None
        if has_prev:
            hprev_ref = refs[k]
            k += 1
        if has_next:
            wn_ref = refs[k]
            k += 1
        h_ref = refs[k]
        k += 1
        hs_ref = refs[k] if has_next else None

        dinvp = dinvp_ref[...]
        m64 = m64_ref[...]
        v = agg_ref[...] * dinvp + b2_ref[0]
        mu = jnp.dot(v, m64, preferred_element_type=jnp.float32) * (1.0 / 64.0)
        xc = v - mu
        var = jnp.dot(xc * xc, m64,
                      preferred_element_type=jnp.float32) * (1.0 / 64.0)
        y = xc * lax.rsqrt(var + 1e-5) * g2_ref[0] + beta2_ref[0]
        y = jnp.where(y >= 0, y, 0.01 * y)
        hcur = 0.5 * y
        if has_prev:
            hcur = hcur + hprev_ref[...]
        h_ref[...] = hcur
        if has_next:
            hs_ref[...] = jnp.dot(
                y, wn_ref[...], preferred_element_type=jnp.float32) * dinvp

    return body


def _post_call(agg2, dinvp, m64, b2, g2, beta2, hprev, w2next):
    has_prev = hprev is not None
    has_next = w2next is not None
    in_specs = [
        pl.BlockSpec((_BR, 128), lambda i: (i, 0)),
        pl.BlockSpec((_BR, 128), lambda i: (i, 0)),
        pl.BlockSpec((128, 128), lambda i: (0, 0)),
        pl.BlockSpec((1, 128), lambda i: (0, 0)),
        pl.BlockSpec((1, 128), lambda i: (0, 0)),
        pl.BlockSpec((1, 128), lambda i: (0, 0)),
    ]
    args = [agg2, dinvp, m64, b2, g2, beta2]
    if has_prev:
        in_specs.append(pl.BlockSpec((_BR, 128), lambda i: (i, 0)))
        args.append(hprev)
    if has_next:
        in_specs.append(pl.BlockSpec((128, 128), lambda i: (0, 0)))
        args.append(w2next)
    out_specs = [pl.BlockSpec((_BR, 128), lambda i: (i, 0))]
    out_shape = [jax.ShapeDtypeStruct((NPAD // 2, 128), jnp.float32)]
    if has_next:
        out_specs.append(pl.BlockSpec((_BR, 128), lambda i: (i, 0)))
        out_shape.append(jax.ShapeDtypeStruct((NPAD // 2, 128), jnp.float32))
    res = pl.pallas_call(
        _make_post_body(has_prev, has_next),
        grid=_GRID,
        in_specs=in_specs,
        out_specs=out_specs,
        out_shape=out_shape,
    )(*args)
    return res if has_next else (res[0], None)


def kernel(x, edge_index, W0, b0, g0, beta0, W1, b1, g1, beta1,
           W2, b2, g2, beta2, W3, b3, g3, beta3):
    Ws = [W0, W1, W2, W3]
    bs = [b0, b1, b2, b3]
    gs = [g0, g1, g2, g3]
    betas = [beta0, beta1, beta2, beta3]

    src = edge_index[0]
    dst = edge_index[1]
    loop_ids = jnp.arange(N, dtype=jnp.int32)
    pad_ids = N + (jnp.arange(EPAD - E - N, dtype=jnp.int32) % (NPAD - N))
    src_p = jnp.concatenate([src, loop_ids, pad_ids])
    dst_p = jnp.concatenate([dst, loop_ids, pad_ids])
    srcg = jnp.stack([2 * src_p, 2 * src_p + 1]).reshape(2, S, _W)
    dsts = dst_p.reshape(S, _W)
    x2 = jnp.pad(x, ((0, NPAD - N), (0, 0))).reshape(NPAD // 2, 128)

    zero128 = jnp.zeros((128, 128), jnp.float32)
    w2s = [jnp.block([[w, jnp.zeros((D, D), w.dtype)],
                      [jnp.zeros((D, D), w.dtype), w]]) for w in Ws]
    ones64 = jnp.ones((D, D), jnp.float32)
    m64 = zero128.at[:D, :D].set(ones64).at[D:, D:].set(ones64)
    sel = jnp.zeros((16, 128), jnp.float32).at[0, :D].set(1.0).at[8, D:].set(1.0)
    b2s = [jnp.tile(b, 2).reshape(1, 128) for b in bs]
    g2s = [jnp.tile(g, 2).reshape(1, 128) for g in gs]
    beta2s = [jnp.tile(bb, 2).reshape(1, 128) for bb in betas]

    zrows = jnp.zeros((RPT, 8), jnp.float32)
    zrows32 = jnp.zeros((RPT, 32), jnp.float32)
    upat = jnp.zeros((_W, 8), jnp.float32).at[:, 0].set(1.0)
    degp2 = _deg_kernel(dsts, zrows, upat).reshape(2, NPAD // 2, 16)
    hs, dinvp = _pre_call(x2, degp2, w2s[0], sel)
    h = None
    for i in range(4):
        agg = _agg_kernel(hs.reshape(2 * NPAD, 32), zrows32, srcg, dsts)
        agg2 = agg.reshape(NPAD // 2, 128)
        w2next = w2s[i + 1] if i < 3 else None
        h, hs = _post_call(agg2, dinvp, m64, b2s[i], g2s[i], beta2s[i], h,
                           w2next)
    return h.reshape(NPAD, D)[:N]


# speedup vs baseline: 1.2882x; 1.0465x over previous
import functools

import jax
import jax.numpy as jnp
from jax import lax
from jax.experimental import pallas as pl
from jax.experimental.pallas import tpu as pltpu
from jax.experimental.pallas import tpu_sc as plsc

N = 50000
E = 800000
D = 64

NPAD = 51200
RPT = NPAD // 16
_W = 96
EPAD = 860160
S = EPAD // _W
SPT = S // 16
SPT_HALF = S // 32

_SC_MESH = plsc.VectorSubcoreMesh(core_axis_name="c", subcore_axis_name="s")
_SC_PARAMS = pltpu.CompilerParams(use_tc_tiling_on_sc=False)


_DG = 7
_DNGRP = SPT_HALF // _DG


@functools.partial(
    pl.kernel,
    out_type=jax.ShapeDtypeStruct((2, 16, RPT, 8), jnp.float32),
    mesh=_SC_MESH,
    compiler_params=_SC_PARAMS,
    scratch_types=[
        pltpu.VMEM((_W, 8), jnp.float32),
        pltpu.VMEM((2, _DG, _W), jnp.int32),
        pltpu.VMEM_SHARED((NPAD, 8), jnp.float32),
        pltpu.SemaphoreType.DMA,
    ],
)
def _deg_kernel(dst_hbm, zrows_hbm, upat_hbm, out_hbm, upat_v, didx, acc,
                ssem):
    c = lax.axis_index("c")
    s = lax.axis_index("s")
    pltpu.sync_copy(upat_hbm, upat_v)
    base = s * RPT
    pltpu.sync_copy(zrows_hbm, acc.at[pl.ds(base, RPT)])
    plsc.subcore_barrier()

    t0 = c * (S // 2) + s * SPT_HALF

    def stage(grp, b):
        pltpu.sync_copy(dst_hbm.at[pl.ds(t0 + grp * _DG, _DG)], didx.at[b])

    stage(0, 0)

    @pl.loop(0, _DNGRP, step=2)
    def _(g):
        for b in range(2):
            gg = g + b

            @pl.when(gg + 1 < _DNGRP)
            def _():
                stage(gg + 1, 1 - b)

            for j in range(_DG):
                pltpu.async_copy(upat_v, acc.at[didx.at[b, j]], ssem,
                                 add=True)
            for j in range(_DG):
                pltpu.make_async_copy(upat_v, acc.at[didx.at[b, j]],
                                      ssem).wait()

    plsc.subcore_barrier()
    pltpu.sync_copy(acc.at[pl.ds(base, RPT)], out_hbm.at[c, s])


_G = 4
_NGRP = SPT // _G


@functools.partial(
    pl.kernel,
    out_type=jax.ShapeDtypeStruct((NPAD, 2, 32), jnp.float32),
    mesh=_SC_MESH,
    compiler_params=_SC_PARAMS,
    scratch_types=[
        pltpu.VMEM((4, _G, _W), jnp.int32),
        pltpu.VMEM((4, _G, _W), jnp.int32),
        pltpu.VMEM((2, _G, _W, 32), jnp.float32),
        pltpu.VMEM_SHARED((NPAD, 32), jnp.float32),
        pltpu.SemaphoreType.DMA((4,)),
        pltpu.SemaphoreType.DMA((2,)),
        pltpu.SemaphoreType.DMA,
    ],
)
def _agg_kernel(hs_hbm, zrows_hbm, srcg_hbm, dst_hbm, out_hbm, sidx, didx,
                rows, acc, isem, gsem, ssem):
    c = lax.axis_index("c")
    s = lax.axis_index("s")
    base = s * RPT
    pltpu.sync_copy(zrows_hbm, acc.at[pl.ds(base, RPT)])
    plsc.subcore_barrier()

    t0 = s * SPT

    def stage_async(grp, ib):
        pltpu.async_copy(srcg_hbm.at[c, pl.ds(t0 + grp * _G, _G)],
                         sidx.at[ib], isem.at[ib])
        pltpu.async_copy(dst_hbm.at[pl.ds(t0 + grp * _G, _G)],
                         didx.at[ib], isem.at[ib])

    def wait_stage(grp, ib):
        pltpu.make_async_copy(srcg_hbm.at[c, pl.ds(t0 + grp * _G, _G)],
                              sidx.at[ib], isem.at[ib]).wait()
        pltpu.make_async_copy(dst_hbm.at[pl.ds(t0 + grp * _G, _G)],
                              didx.at[ib], isem.at[ib]).wait()

    def fire_gathers(ib, rb):
        for j in range(_G):
            pltpu.async_copy(hs_hbm.at[sidx.at[ib, j]], rows.at[rb, j],
                             gsem.at[rb])

    def wait_gathers(ib, rb):
        for j in range(_G):
            pltpu.make_async_copy(hs_hbm.at[sidx.at[ib, j]], rows.at[rb, j],
                                  gsem.at[rb]).wait()

    def fire_scatters(ib, rb):
        for j in range(_G):
            pltpu.async_copy(rows.at[rb, j], acc.at[didx.at[ib, j]], ssem,
                             add=True)

    def drain_scatters(ib, rb):
        for j in range(_G):
            pltpu.make_async_copy(rows.at[rb, j], acc.at[didx.at[ib, j]],
                                  ssem).wait()

    stage_async(0, 0)
    stage_async(1, 1)
    wait_stage(0, 0)
    fire_gathers(0, 0)

    @pl.loop(0, _NGRP, step=4)
    def _(g):
        for b in range(4):
            gg = g + b
            ib = b
            rb = b % 2
            ibp = (b - 1) % 4
            rbp = (b + 1) % 2

            @pl.when(gg >= 1)
            def _():
                drain_scatters(ibp, rbp)

            @pl.when(gg + 2 < _NGRP)
            def _():
                stage_async(gg + 2, (b + 2) % 4)

            @pl.when(gg + 1 < _NGRP)
            def _():
                wait_stage(gg + 1, (b + 1) % 4)
                fire_gathers((b + 1) % 4, (b + 1) % 2)

            wait_gathers(ib, rb)
            fire_scatters(ib, rb)

    drain_scatters(3, 1)
    plsc.subcore_barrier()
    pltpu.sync_copy(acc.at[pl.ds(base, RPT)],
                    out_hbm.at[pl.ds(base, RPT), c])


_BN = 5120
_BR = _BN // 2
_GRID = (NPAD // _BN,)


def _pre_body(x_ref, degp_ref, w2_ref, sel_ref, hs_ref, dinvp_ref):
    dd = degp_ref[...]
    d0 = dd[0] + dd[1]
    dinvp = lax.rsqrt(jnp.maximum(
        jnp.dot(d0, sel_ref[...], preferred_element_type=jnp.float32), 1.0))
    h = jnp.dot(x_ref[...], w2_ref[...], preferred_element_type=jnp.float32)
    hs_ref[...] = h * dinvp
    dinvp_ref[...] = dinvp


def _pre_call(x2, degp2, w2, sel):
    return pl.pallas_call(
        _pre_body,
        grid=_GRID,
        in_specs=[
            pl.BlockSpec((_BR, 128), lambda i: (i, 0)),
            pl.BlockSpec((2, _BR, 16), lambda i: (0, i, 0)),
            pl.BlockSpec((128, 128), lambda i: (0, 0)),
            pl.BlockSpec((16, 128), lambda i: (0, 0)),
        ],
        out_specs=[
            pl.BlockSpec((_BR, 128), lambda i: (i, 0)),
            pl.BlockSpec((_BR, 128), lambda i: (i, 0)),
        ],
        out_shape=[
            jax.ShapeDtypeStruct((NPAD // 2, 128), jnp.float32),
            jax.ShapeDtypeStruct((NPAD // 2, 128), jnp.float32),
        ],
    )(x2, degp2, w2, sel)


def _make_post_body(has_prev, has_next):
    def body(*refs):
        agg_ref, dinvp_ref, m64_ref, b2_ref, g2_ref, beta2_ref = refs[:6]
        k = 6
        hprev_ref = None
        wn_ref = None
        if has_prev:
            hprev_ref = refs[k]
            k += 1
        if has_next:
            wn_ref = refs[k]
            k += 1
        h_ref = refs[k]
        k += 1
        hs_ref = refs[k] if has_next else None

        dinvp = dinvp_ref[...]
        m64 = m64_ref[...]
        v = agg_ref[...] * dinvp + b2_ref[0]
        mu = jnp.dot(v, m64, preferred_element_type=jnp.float32) * (1.0 / 64.0)
        xc = v - mu
        var = jnp.dot(xc * xc, m64,
                      preferred_element_type=jnp.float32) * (1.0 / 64.0)
        y = xc * lax.rsqrt(var + 1e-5) * g2_ref[0] + beta2_ref[0]
        y = jnp.where(y >= 0, y, 0.01 * y)
        hcur = 0.5 * y
        if has_prev:
            hcur = hcur + hprev_ref[...]
        h_ref[...] = hcur
        if has_next:
            hs_ref[...] = jnp.dot(
                y, wn_ref[...], preferred_element_type=jnp.float32) * dinvp

    return body


def _post_call(agg2, dinvp, m64, b2, g2, beta2, hprev, w2next):
    has_prev = hprev is not None
    has_next = w2next is not None
    in_specs = [
        pl.BlockSpec((_BR, 128), lambda i: (i, 0)),
        pl.BlockSpec((_BR, 128), lambda i: (i, 0)),
        pl.BlockSpec((128, 128), lambda i: (0, 0)),
        pl.BlockSpec((1, 128), lambda i: (0, 0)),
        pl.BlockSpec((1, 128), lambda i: (0, 0)),
        pl.BlockSpec((1, 128), lambda i: (0, 0)),
    ]
    args = [agg2, dinvp, m64, b2, g2, beta2]
    if has_prev:
        in_specs.append(pl.BlockSpec((_BR, 128), lambda i: (i, 0)))
        args.append(hprev)
    if has_next:
        in_specs.append(pl.BlockSpec((128, 128), lambda i: (0, 0)))
        args.append(w2next)
    out_specs = [pl.BlockSpec((_BR, 128), lambda i: (i, 0))]
    out_shape = [jax.ShapeDtypeStruct((NPAD // 2, 128), jnp.float32)]
    if has_next:
        out_specs.append(pl.BlockSpec((_BR, 128), lambda i: (i, 0)))
        out_shape.append(jax.ShapeDtypeStruct((NPAD // 2, 128), jnp.float32))
    res = pl.pallas_call(
        _make_post_body(has_prev, has_next),
        grid=_GRID,
        in_specs=in_specs,
        out_specs=out_specs,
        out_shape=out_shape,
    )(*args)
    return res if has_next else (res[0], None)


def kernel(x, edge_index, W0, b0, g0, beta0, W1, b1, g1, beta1,
           W2, b2, g2, beta2, W3, b3, g3, beta3):
    Ws = [W0, W1, W2, W3]
    bs = [b0, b1, b2, b3]
    gs = [g0, g1, g2, g3]
    betas = [beta0, beta1, beta2, beta3]

    src = edge_index[0]
    dst = edge_index[1]
    loop_ids = jnp.arange(N, dtype=jnp.int32)
    pad_ids = N + (jnp.arange(EPAD - E - N, dtype=jnp.int32) % (NPAD - N))
    src_p = jnp.concatenate([src, loop_ids, pad_ids])
    dst_p = jnp.concatenate([dst, loop_ids, pad_ids])
    srcg = jnp.stack([2 * src_p, 2 * src_p + 1]).reshape(2, S, _W)
    dsts = dst_p.reshape(S, _W)
    x2 = jnp.pad(x, ((0, NPAD - N), (0, 0))).reshape(NPAD // 2, 128)

    zero128 = jnp.zeros((128, 128), jnp.float32)
    w2s = [jnp.block([[w, jnp.zeros((D, D), w.dtype)],
                      [jnp.zeros((D, D), w.dtype), w]]) for w in Ws]
    ones64 = jnp.ones((D, D), jnp.float32)
    m64 = zero128.at[:D, :D].set(ones64).at[D:, D:].set(ones64)
    sel = jnp.zeros((16, 128), jnp.float32).at[0, :D].set(1.0).at[8, D:].set(1.0)
    b2s = [jnp.tile(b, 2).reshape(1, 128) for b in bs]
    g2s = [jnp.tile(g, 2).reshape(1, 128) for g in gs]
    beta2s = [jnp.tile(bb, 2).reshape(1, 128) for bb in betas]

    zrows = jnp.zeros((RPT, 8), jnp.float32)
    zrows32 = jnp.zeros((RPT, 32), jnp.float32)
    upat = jnp.zeros((_W, 8), jnp.float32).at[:, 0].set(1.0)
    degp2 = _deg_kernel(dsts, zrows, upat).reshape(2, NPAD // 2, 16)
    hs, dinvp = _pre_call(x2, degp2, w2s[0], sel)
    h = None
    for i in range(4):
        agg = _agg_kernel(hs.reshape(2 * NPAD, 32), zrows32, srcg, dsts)
        agg2 = agg.reshape(NPAD // 2, 128)
        w2next = w2s[i + 1] if i < 3 else None
        h, hs = _post_call(agg2, dinvp, m64, b2s[i], g2s[i], beta2s[i], h,
                           w2next)
    return h.reshape(NPAD, D)[:N]


# speedup vs baseline: 1.2931x; 1.0038x over previous
import functools

import jax
import jax.numpy as jnp
from jax import lax
from jax.experimental import pallas as pl
from jax.experimental.pallas import tpu as pltpu
from jax.experimental.pallas import tpu_sc as plsc

N = 50000
E = 800000
D = 64

NPAD = 51200
RPT = NPAD // 16
_W = 96
EPAD = 860160
S = EPAD // _W
SPT = S // 16
SPT_HALF = S // 32

_SC_MESH = plsc.VectorSubcoreMesh(core_axis_name="c", subcore_axis_name="s")
_SC_PARAMS = pltpu.CompilerParams(use_tc_tiling_on_sc=False)


_DG = 7
_DNGRP = SPT_HALF // _DG


@functools.partial(
    pl.kernel,
    out_type=jax.ShapeDtypeStruct((2, 16, RPT, 8), jnp.float32),
    mesh=_SC_MESH,
    compiler_params=_SC_PARAMS,
    scratch_types=[
        pltpu.VMEM((_W, 8), jnp.float32),
        pltpu.VMEM((2, _DG, _W), jnp.int32),
        pltpu.VMEM_SHARED((NPAD, 8), jnp.float32),
        pltpu.SemaphoreType.DMA,
    ],
)
def _deg_kernel(dst_hbm, zrows_hbm, upat_hbm, out_hbm, upat_v, didx, acc,
                ssem):
    c = lax.axis_index("c")
    s = lax.axis_index("s")
    pltpu.sync_copy(upat_hbm, upat_v)
    base = s * RPT
    pltpu.sync_copy(zrows_hbm, acc.at[pl.ds(base, RPT)])
    plsc.subcore_barrier()

    t0 = c * (S // 2) + s * SPT_HALF

    def stage(grp, b):
        pltpu.sync_copy(dst_hbm.at[pl.ds(t0 + grp * _DG, _DG)], didx.at[b])

    stage(0, 0)

    @pl.loop(0, _DNGRP, step=2)
    def _(g):
        for b in range(2):
            gg = g + b

            @pl.when(gg + 1 < _DNGRP)
            def _():
                stage(gg + 1, 1 - b)

            for j in range(_DG):
                pltpu.async_copy(upat_v, acc.at[didx.at[b, j]], ssem,
                                 add=True)
            for j in range(_DG):
                pltpu.make_async_copy(upat_v, acc.at[didx.at[b, j]],
                                      ssem).wait()

    plsc.subcore_barrier()
    pltpu.sync_copy(acc.at[pl.ds(base, RPT)], out_hbm.at[c, s])


_G = 4
_NGRP = SPT // _G


@functools.partial(
    pl.kernel,
    out_type=jax.ShapeDtypeStruct((NPAD, 2, 32), jnp.float32),
    mesh=_SC_MESH,
    compiler_params=_SC_PARAMS,
    scratch_types=[
        pltpu.VMEM((4, _G, _W), jnp.int32),
        pltpu.VMEM((4, _G, _W), jnp.int32),
        pltpu.VMEM((2, _G, _W, 32), jnp.float32),
        pltpu.VMEM_SHARED((NPAD, 32), jnp.float32),
        pltpu.SemaphoreType.DMA((4,)),
        pltpu.SemaphoreType.DMA((2,)),
        pltpu.SemaphoreType.DMA,
    ],
)
def _agg_kernel(hs_hbm, zrows_hbm, srcg_hbm, dst_hbm, out_hbm, sidx, didx,
                rows, acc, isem, gsem, ssem):
    c = lax.axis_index("c")
    s = lax.axis_index("s")
    base = s * RPT
    pltpu.sync_copy(zrows_hbm, acc.at[pl.ds(base, RPT)])
    plsc.subcore_barrier()

    t0 = s * SPT

    def stage_async(grp, ib):
        pltpu.async_copy(srcg_hbm.at[c, pl.ds(t0 + grp * _G, _G)],
                         sidx.at[ib], isem.at[ib])
        pltpu.async_copy(dst_hbm.at[pl.ds(t0 + grp * _G, _G)],
                         didx.at[ib], isem.at[ib])

    def wait_stage(grp, ib):
        pltpu.make_async_copy(srcg_hbm.at[c, pl.ds(t0 + grp * _G, _G)],
                              sidx.at[ib], isem.at[ib]).wait()
        pltpu.make_async_copy(dst_hbm.at[pl.ds(t0 + grp * _G, _G)],
                              didx.at[ib], isem.at[ib]).wait()

    def fire_gathers(ib, rb):
        for j in range(_G):
            pltpu.async_copy(hs_hbm.at[sidx.at[ib, j]], rows.at[rb, j],
                             gsem.at[rb])

    def wait_gathers(ib, rb):
        for j in range(_G):
            pltpu.make_async_copy(hs_hbm.at[sidx.at[ib, j]], rows.at[rb, j],
                                  gsem.at[rb]).wait()

    def fire_scatters(ib, rb):
        for j in range(_G):
            pltpu.async_copy(rows.at[rb, j], acc.at[didx.at[ib, j]], ssem,
                             add=True)

    def drain_scatters(ib, rb):
        for j in range(_G):
            pltpu.make_async_copy(rows.at[rb, j], acc.at[didx.at[ib, j]],
                                  ssem).wait()

    stage_async(0, 0)
    stage_async(1, 1)
    wait_stage(0, 0)
    fire_gathers(0, 0)

    @pl.loop(0, _NGRP, step=4)
    def _(g):
        for b in range(4):
            gg = g + b
            ib = b
            rb = b % 2
            ibp = (b - 1) % 4
            rbp = (b + 1) % 2

            @pl.when(gg >= 1)
            def _():
                drain_scatters(ibp, rbp)

            @pl.when(gg + 2 < _NGRP)
            def _():
                stage_async(gg + 2, (b + 2) % 4)

            @pl.when(gg + 1 < _NGRP)
            def _():
                wait_stage(gg + 1, (b + 1) % 4)
                fire_gathers((b + 1) % 4, (b + 1) % 2)

            wait_gathers(ib, rb)
            fire_scatters(ib, rb)

    drain_scatters(3, 1)
    plsc.subcore_barrier()
    pltpu.sync_copy(acc.at[pl.ds(base, RPT)],
                    out_hbm.at[pl.ds(base, RPT), c])


_BN = 10240
_BR = _BN // 2
_GRID = (NPAD // _BN,)


def _pre_body(x_ref, degp_ref, w2_ref, sel_ref, hs_ref, dinvp_ref):
    dd = degp_ref[...]
    d0 = dd[0] + dd[1]
    dinvp = lax.rsqrt(jnp.maximum(
        jnp.dot(d0, sel_ref[...], preferred_element_type=jnp.float32), 1.0))
    h = jnp.dot(x_ref[...], w2_ref[...], preferred_element_type=jnp.float32)
    hs_ref[...] = h * dinvp
    dinvp_ref[...] = dinvp


def _pre_call(x2, degp2, w2, sel):
    return pl.pallas_call(
        _pre_body,
        grid=_GRID,
        in_specs=[
            pl.BlockSpec((_BR, 128), lambda i: (i, 0)),
            pl.BlockSpec((2, _BR, 16), lambda i: (0, i, 0)),
            pl.BlockSpec((128, 128), lambda i: (0, 0)),
            pl.BlockSpec((16, 128), lambda i: (0, 0)),
        ],
        out_specs=[
            pl.BlockSpec((_BR, 128), lambda i: (i, 0)),
            pl.BlockSpec((_BR, 128), lambda i: (i, 0)),
        ],
        out_shape=[
            jax.ShapeDtypeStruct((NPAD // 2, 128), jnp.float32),
            jax.ShapeDtypeStruct((NPAD // 2, 128), jnp.float32),
        ],
    )(x2, degp2, w2, sel)


def _make_post_body(has_prev, has_next):
    def body(*refs):
        agg_ref, dinvp_ref, m64_ref, b2_ref, g2_ref, beta2_ref = refs[:6]
        k = 6
        hprev_ref = None
        wn_ref = None
        if has_prev:
            hprev_ref = refs[k]
            k += 1
        if has_next:
            wn_ref = refs[k]
            k += 1
        h_ref = refs[k]
        k += 1
        hs_ref = refs[k] if has_next else None

        dinvp = dinvp_ref[...]
        m64 = m64_ref[...]
        v = agg_ref[...] * dinvp + b2_ref[0]
        mu = jnp.dot(v, m64, preferred_element_type=jnp.float32) * (1.0 / 64.0)
        xc = v - mu
        var = jnp.dot(xc * xc, m64,
                      preferred_element_type=jnp.float32) * (1.0 / 64.0)
        y = xc * lax.rsqrt(var + 1e-5) * g2_ref[0] + beta2_ref[0]
        y = jnp.where(y >= 0, y, 0.01 * y)
        hcur = 0.5 * y
        if has_prev:
            hcur = hcur + hprev_ref[...]
        h_ref[...] = hcur
        if has_next:
            hs_ref[...] = jnp.dot(
                y, wn_ref[...], preferred_element_type=jnp.float32) * dinvp

    return body


def _post_call(agg2, dinvp, m64, b2, g2, beta2, hprev, w2next):
    has_prev = hprev is not None
    has_next = w2next is not None
    in_specs = [
        pl.BlockSpec((_BR, 128), lambda i: (i, 0)),
        pl.BlockSpec((_BR, 128), lambda i: (i, 0)),
        pl.BlockSpec((128, 128), lambda i: (0, 0)),
        pl.BlockSpec((1, 128), lambda i: (0, 0)),
        pl.BlockSpec((1, 128), lambda i: (0, 0)),
        pl.BlockSpec((1, 128), lambda i: (0, 0)),
    ]
    args = [agg2, dinvp, m64, b2, g2, beta2]
    if has_prev:
        in_specs.append(pl.BlockSpec((_BR, 128), lambda i: (i, 0)))
        args.append(hprev)
    if has_next:
        in_specs.append(pl.BlockSpec((128, 128), lambda i: (0, 0)))
        args.append(w2next)
    out_specs = [pl.BlockSpec((_BR, 128), lambda i: (i, 0))]
    out_shape = [jax.ShapeDtypeStruct((NPAD // 2, 128), jnp.float32)]
    if has_next:
        out_specs.append(pl.BlockSpec((_BR, 128), lambda i: (i, 0)))
        out_shape.append(jax.ShapeDtypeStruct((NPAD // 2, 128), jnp.float32))
    res = pl.pallas_call(
        _make_post_body(has_prev, has_next),
        grid=_GRID,
        in_specs=in_specs,
        out_specs=out_specs,
        out_shape=out_shape,
    )(*args)
    return res if has_next else (res[0], None)


def kernel(x, edge_index, W0, b0, g0, beta0, W1, b1, g1, beta1,
           W2, b2, g2, beta2, W3, b3, g3, beta3):
    Ws = [W0, W1, W2, W3]
    bs = [b0, b1, b2, b3]
    gs = [g0, g1, g2, g3]
    betas = [beta0, beta1, beta2, beta3]

    src = edge_index[0]
    dst = edge_index[1]
    loop_ids = jnp.arange(N, dtype=jnp.int32)
    pad_ids = N + (jnp.arange(EPAD - E - N, dtype=jnp.int32) % (NPAD - N))
    src_p = jnp.concatenate([src, loop_ids, pad_ids])
    dst_p = jnp.concatenate([dst, loop_ids, pad_ids])
    srcg = jnp.stack([2 * src_p, 2 * src_p + 1]).reshape(2, S, _W)
    dsts = dst_p.reshape(S, _W)
    x2 = jnp.pad(x, ((0, NPAD - N), (0, 0))).reshape(NPAD // 2, 128)

    zero128 = jnp.zeros((128, 128), jnp.float32)
    w2s = [jnp.block([[w, jnp.zeros((D, D), w.dtype)],
                      [jnp.zeros((D, D), w.dtype), w]]) for w in Ws]
    ones64 = jnp.ones((D, D), jnp.float32)
    m64 = zero128.at[:D, :D].set(ones64).at[D:, D:].set(ones64)
    sel = jnp.zeros((16, 128), jnp.float32).at[0, :D].set(1.0).at[8, D:].set(1.0)
    b2s = [jnp.tile(b, 2).reshape(1, 128) for b in bs]
    g2s = [jnp.tile(g, 2).reshape(1, 128) for g in gs]
    beta2s = [jnp.tile(bb, 2).reshape(1, 128) for bb in betas]

    zrows = jnp.zeros((RPT, 8), jnp.float32)
    zrows32 = jnp.zeros((RPT, 32), jnp.float32)
    upat = jnp.zeros((_W, 8), jnp.float32).at[:, 0].set(1.0)
    degp2 = _deg_kernel(dsts, zrows, upat).reshape(2, NPAD // 2, 16)
    hs, dinvp = _pre_call(x2, degp2, w2s[0], sel)
    h = None
    for i in range(4):
        agg = _agg_kernel(hs.reshape(2 * NPAD, 32), zrows32, srcg, dsts)
        agg2 = agg.reshape(NPAD // 2, 128)
        w2next = w2s[i + 1] if i < 3 else None
        h, hs = _post_call(agg2, dinvp, m64, b2s[i], g2s[i], beta2s[i], h,
                           w2next)
    return h.reshape(NPAD, D)[:N]
